# baseline (speedup 1.0000x reference)
import functools

import jax
import jax.numpy as jnp
from jax.experimental import pallas as pl
from jax.experimental.pallas import tpu as pltpu

N = 10000
E = 160000
D = 128
L = 4
NB = 128
NG = 556
H = 4
DH = 32
DMID = 256
DF = 512
CUT = 5.0
AVG_DEG = 15.57930850982666
AVG_NODES = 18.03065905448718

BN = 1000


def _sph(vec):
    r = jnp.linalg.norm(vec, axis=-1, keepdims=True)
    u = vec / (r + 1e-8)
    x, y, z = u[:, 0], u[:, 1], u[:, 2]
    s3 = 3.0 ** 0.5
    s5 = 5.0 ** 0.5
    s15 = 15.0 ** 0.5
    return jnp.stack([jnp.ones_like(x), s3 * x, s3 * y, s3 * z,
                      s15 * x * y, s15 * y * z, (s5 / 2.0) * (3.0 * z * z - 1.0),
                      s15 * x * z, (s15 / 2.0) * (x * x - y * y)], axis=-1)


def _rbf(r):
    centers = jnp.linspace(0.0, CUT, NB)
    width = CUT / NB
    return jnp.exp(-(((r[:, None] - centers[None, :]) / width) ** 2))


def _head_body(x_ref, wproj_ref, h1_ref, h2_ref, o_ref):
    x = x_ref[...]
    xp = jnp.dot(x, wproj_ref[...], preferred_element_type=jnp.float32)
    mu = xp.mean(axis=-1, keepdims=True)
    sd = jnp.sqrt(((xp - mu) ** 2).mean(axis=-1, keepdims=True) + 1e-5)
    xn = (xp - mu) / sd
    h = jnp.dot(xn, h1_ref[...], preferred_element_type=jnp.float32)
    h = h * jax.nn.sigmoid(h)
    o_ref[...] = jnp.dot(h, h2_ref[...], preferred_element_type=jnp.float32)


def _head(x, Wproj, head_w1, head_w2):
    return pl.pallas_call(
        _head_body,
        grid=(N // BN,),
        in_specs=[
            pl.BlockSpec((BN, D), lambda i: (i, 0)),
            pl.BlockSpec((D, DF), lambda i: (0, 0)),
            pl.BlockSpec((DF, DF), lambda i: (0, 0)),
            pl.BlockSpec((DF, 128), lambda i: (0, 0)),
        ],
        out_specs=pl.BlockSpec((BN, 128), lambda i: (i, 0)),
        out_shape=jax.ShapeDtypeStruct((N, 128), jnp.float32),
    )(x, Wproj, head_w1, head_w2)


def kernel(f_in, pos, batch, node_atom, edge_src, edge_dst, atom_table,
           deg_w1, deg_w2, deg_w3, Wq, Wk, Wv, Wo, We1, We2, Wsh, Wf1, Wf2,
           Wproj, head_w1, head_w2):
    edge_vec = pos[edge_src] - pos[edge_dst]
    sh = _sph(edge_vec)
    r = jnp.linalg.norm(edge_vec, axis=-1)
    rbf = _rbf(r)
    x0 = atom_table[f_in]
    g = jax.nn.silu(rbf @ deg_w1)
    g = jax.nn.silu(g @ deg_w2)
    gate = g @ deg_w3
    msg = x0[edge_src] * gate * sh[:, :1]
    deg_emb = jax.ops.segment_sum(msg, edge_dst, num_segments=N) / (AVG_DEG ** 0.5)
    x = x0 + deg_emb
    for l in range(L):
        ef = jax.nn.silu(rbf @ We1[l]) @ We2[l] + sh @ Wsh[l]
        q = (x @ Wq[l])[edge_dst].reshape(E, H, DH)
        src_f = x[edge_src] + ef
        k = (src_f @ Wk[l]).reshape(E, H, DH)
        v = (src_f @ Wv[l]).reshape(E, H, DH)
        logits = (q * k).sum(-1) / (DH ** 0.5)
        m = jax.ops.segment_max(logits, edge_dst, num_segments=N)
        m = jnp.where(jnp.isfinite(m), m, 0.0)
        a = jnp.exp(logits - m[edge_dst])
        denom = jax.ops.segment_sum(a, edge_dst, num_segments=N)
        a = a / (denom[edge_dst] + 1e-9)
        agg = jax.ops.segment_sum((a[:, :, None] * v).reshape(E, D), edge_dst, num_segments=N)
        x = x + agg @ Wo[l]
        x = x + jax.nn.silu(x @ Wf1[l]) @ Wf2[l]
    h2p = jnp.pad(head_w2, ((0, 0), (0, 127)))
    node_out = _head(x, Wproj, head_w1, h2p)[:, :1]
    out = jax.ops.segment_sum(node_out, batch, num_segments=NG) / (AVG_NODES ** 0.5)
    return out


# speedup vs baseline: 2.5619x; 2.5619x over previous
import functools

import jax
import jax.numpy as jnp
from jax import lax
from jax.experimental import pallas as pl
from jax.experimental.pallas import tpu as pltpu
from jax.experimental.pallas import tpu_sc as plsc

N = 10000
E = 160000
D = 128
L = 4
NB = 128
NG = 556
H = 4
DH = 32
DMID = 256
DF = 512
CUT = 5.0
AVG_DEG = 15.57930850982666
AVG_NODES = 18.03065905448718

NC = 2
NS = 16
NW = NC * NS

KE, CE = 125, 40
NP = 10240
KN, CN = 4, 80

NR_NODE = 10112
NR_HEAD = 640


def _pack_idx(idx, K, C):
    return idx.reshape(NW, K, C)


def _sc_gather(table, idx3, K, C, Dp):
    B = NW * K * C
    mesh = plsc.VectorSubcoreMesh(core_axis_name="c", subcore_axis_name="s")

    @functools.partial(
        pl.kernel,
        out_type=jax.ShapeDtypeStruct((B, Dp), jnp.float32),
        mesh=mesh,
        scratch_types=[
            pltpu.VMEM((K, C), jnp.int32),
            pltpu.VMEM((C, Dp), jnp.float32),
            pltpu.SemaphoreType.DMA,
        ],
    )
    def gk(table_hbm, idx_hbm, out_hbm, idxv, rows, sem):
        w = lax.axis_index("s") * NC + lax.axis_index("c")
        pltpu.sync_copy(idx_hbm.at[w], idxv)

        def chunk(j, carry):
            pltpu.async_copy(table_hbm.at[idxv.at[j]], rows, sem).wait()
            pltpu.sync_copy(rows, out_hbm.at[pl.ds(w * (K * C) + j * C, C)])
            return carry

        lax.fori_loop(0, K, chunk, 0)

    return gk(table, idx3)


def _sc_scatter(vals, idx3, K, C, Dp, NR):
    NRs = NR // NS
    zeros = jnp.zeros((NRs, Dp), jnp.float32)
    mesh = plsc.VectorSubcoreMesh(core_axis_name="c", subcore_axis_name="s")

    @functools.partial(
        pl.kernel,
        out_type=jax.ShapeDtypeStruct((NC, NR, Dp), jnp.float32),
        mesh=mesh,
        scratch_types=[
            pltpu.VMEM_SHARED((NR, Dp), jnp.float32),
            pltpu.VMEM((K, C), jnp.int32),
            pltpu.VMEM((C, Dp), jnp.float32),
        ],
    )
    def sk(vals_hbm, idx_hbm, zeros_hbm, out_hbm, acc, idxv, rows):
        c = lax.axis_index("c")
        s = lax.axis_index("s")
        w = s * NC + c
        pltpu.sync_copy(zeros_hbm, acc.at[pl.ds(s * NRs, NRs)])
        pltpu.sync_copy(idx_hbm.at[w], idxv)
        plsc.subcore_barrier()

        def chunk(j, carry):
            pltpu.sync_copy(vals_hbm.at[pl.ds(w * (K * C) + j * C, C)], rows)
            pltpu.sync_copy(rows, acc.at[idxv.at[j]], add=True)
            return carry

        lax.fori_loop(0, K, chunk, 0)
        plsc.subcore_barrier()
        pltpu.sync_copy(acc.at[pl.ds(s * NRs, NRs)],
                        out_hbm.at[c, pl.ds(s * NRs, NRs)])

    return sk(vals, idx3, zeros)


def _sph(vec):
    r = jnp.linalg.norm(vec, axis=-1, keepdims=True)
    u = vec / (r + 1e-8)
    x, y, z = u[:, 0], u[:, 1], u[:, 2]
    s3 = 3.0 ** 0.5
    s5 = 5.0 ** 0.5
    s15 = 15.0 ** 0.5
    return jnp.stack([jnp.ones_like(x), s3 * x, s3 * y, s3 * z,
                      s15 * x * y, s15 * y * z, (s5 / 2.0) * (3.0 * z * z - 1.0),
                      s15 * x * z, (s15 / 2.0) * (x * x - y * y)], axis=-1)


def _rbf(r):
    centers = jnp.linspace(0.0, CUT, NB)
    width = CUT / NB
    return jnp.exp(-(((r[:, None] - centers[None, :]) / width) ** 2))


BN = 1000


def _head_body(x_ref, wproj_ref, h1_ref, h2_ref, o_ref):
    x = x_ref[...]
    xp = jnp.dot(x, wproj_ref[...], preferred_element_type=jnp.float32)
    mu = xp.mean(axis=-1, keepdims=True)
    sd = jnp.sqrt(((xp - mu) ** 2).mean(axis=-1, keepdims=True) + 1e-5)
    xn = (xp - mu) / sd
    h = jnp.dot(xn, h1_ref[...], preferred_element_type=jnp.float32)
    h = h * jax.nn.sigmoid(h)
    o_ref[...] = jnp.dot(h, h2_ref[...], preferred_element_type=jnp.float32)


def _head(x, Wproj, head_w1, head_w2p):
    return pl.pallas_call(
        _head_body,
        grid=(N // BN,),
        in_specs=[
            pl.BlockSpec((BN, D), lambda i: (i, 0)),
            pl.BlockSpec((D, DF), lambda i: (0, 0)),
            pl.BlockSpec((DF, DF), lambda i: (0, 0)),
            pl.BlockSpec((DF, 128), lambda i: (0, 0)),
        ],
        out_specs=pl.BlockSpec((BN, 128), lambda i: (i, 0)),
        out_shape=jax.ShapeDtypeStruct((N, 128), jnp.float32),
    )(x, Wproj, head_w1, head_w2p)


def kernel(f_in, pos, batch, node_atom, edge_src, edge_dst, atom_table,
           deg_w1, deg_w2, deg_w3, Wq, Wk, Wv, Wo, We1, We2, Wsh, Wf1, Wf2,
           Wproj, head_w1, head_w2):
    es = edge_src.astype(jnp.int32)
    ed = edge_dst.astype(jnp.int32)
    src3g = _pack_idx(es, KE, CE)
    dst3g = _pack_idx(ed, KE, CE)

    pos_p = jnp.pad(pos, ((0, 0), (0, 125)))
    ps = _sc_gather(pos_p, src3g, KE, CE, 128)
    pd = _sc_gather(pos_p, dst3g, KE, CE, 128)
    edge_vec = (ps - pd)[:, :3]
    sh = _sph(edge_vec)
    r = jnp.linalg.norm(edge_vec, axis=-1)
    rbf = _rbf(r)

    f3 = _pack_idx(jnp.pad(f_in.astype(jnp.int32), (0, NP - N)), KN, CN)
    x0 = _sc_gather(atom_table, f3, KN, CN, D)[:N]

    g = jax.nn.silu(rbf @ deg_w1)
    g = jax.nn.silu(g @ deg_w2)
    gate = g @ deg_w3
    x0s = _sc_gather(x0, src3g, KE, CE, D)
    msg = x0s * gate
    p = _sc_scatter(msg, dst3g, KE, CE, D, NR_NODE)
    deg_emb = (p[0] + p[1])[:N] / (AVG_DEG ** 0.5)
    x = x0 + deg_emb

    for l in range(L):
        ef = jax.nn.silu(rbf @ We1[l]) @ We2[l] + sh @ Wsh[l]
        xq = x @ Wq[l]
        qd = _sc_gather(xq, dst3g, KE, CE, D)
        xs = _sc_gather(x, src3g, KE, CE, D)
        src_f = xs + ef
        k = src_f @ Wk[l]
        v = src_f @ Wv[l]
        logits = (qd.reshape(E, H, DH) * k.reshape(E, H, DH)).sum(-1) / (DH ** 0.5)
        gm = logits.max(axis=0)
        a = jnp.exp(logits - gm[None, :])
        av = (a[:, :, None] * v.reshape(E, H, DH)).reshape(E, D)
        p = _sc_scatter(av, dst3g, KE, CE, D, NR_NODE)
        pd_ = _sc_scatter(jnp.pad(a, ((0, 0), (0, D - H))), dst3g, KE, CE, D, NR_NODE)
        S = (p[0] + p[1])[:N]
        den = (pd_[0] + pd_[1])[:N, :H].reshape(N, H, 1)
        agg = jnp.where(den > 0, S.reshape(N, H, DH) / jnp.where(den > 0, den, 1.0), 0.0)
        x = x + agg.reshape(N, D) @ Wo[l]
        x = x + jax.nn.silu(x @ Wf1[l]) @ Wf2[l]

    h2p = jnp.pad(head_w2, ((0, 0), (0, 127)))
    node_out = _head(x, Wproj, head_w1, h2p)

    no_p = jnp.pad(node_out, ((0, NP - N), (0, 0)))
    b3 = _pack_idx(jnp.pad(batch.astype(jnp.int32), (0, NP - N),
                           constant_values=NR_HEAD - 1), KN, CN)
    hp = _sc_scatter(no_p, b3, KN, CN, D, NR_HEAD)
    out = (hp[0] + hp[1])[:NG, :1] / (AVG_NODES ** 0.5)
    return out


# speedup vs baseline: 3.3237x; 1.2974x over previous
import functools

import jax
import jax.numpy as jnp
from jax import lax
from jax.experimental import pallas as pl
from jax.experimental.pallas import tpu as pltpu
from jax.experimental.pallas import tpu_sc as plsc

N = 10000
E = 160000
D = 128
L = 4
NB = 128
NG = 556
H = 4
DH = 32
DMID = 256
DF = 512
CUT = 5.0
AVG_DEG = 15.57930850982666
AVG_NODES = 18.03065905448718

NC = 2
NS = 16
NW = NC * NS

CHG = 128
NCHG = E // CHG
KWG = NCHG // NW
TAILG = NCHG - KWG * NW
CHS = 64
NCHS = E // CHS
KWS = NCHS // NW
TAILS = NCHS - KWS * NW
GB = 3

NR_NODE = 10112


def _sc_gather(table, idx2):
    mesh = plsc.VectorSubcoreMesh(core_axis_name="c", subcore_axis_name="s")

    @functools.partial(
        pl.kernel,
        out_type=jax.ShapeDtypeStruct((E, D), jnp.float32),
        mesh=mesh,
        scratch_types=[
            pltpu.VMEM((KWG + 1, CHG), jnp.int32),
            pltpu.VMEM((CHG, D), jnp.float32),
            pltpu.VMEM((CHG, D), jnp.float32),
            pltpu.VMEM((CHG, D), jnp.float32),
            pltpu.SemaphoreType.DMA,
            pltpu.SemaphoreType.DMA,
            pltpu.SemaphoreType.DMA,
            pltpu.SemaphoreType.DMA,
            pltpu.SemaphoreType.DMA,
            pltpu.SemaphoreType.DMA,
            pltpu.SemaphoreType.DMA,
        ],
    )
    def gk(table_hbm, idx_hbm, out_hbm, idxv, r0, r1, r2,
           semi, sg0, sg1, sg2, sc0, sc1, sc2):
        rbuf = (r0, r1, r2)
        semg = (sg0, sg1, sg2)
        semc = (sc0, sc1, sc2)
        w = lax.axis_index("s") * NC + lax.axis_index("c")

        def fire_idx(j, carry):
            pltpu.async_copy(idx_hbm.at[w + NW * j], idxv.at[j], semi)
            return carry

        lax.fori_loop(0, KWG, fire_idx, 0)

        def drain_idx(j, carry):
            pltpu.make_async_copy(idx_hbm.at[0], idxv.at[0], semi).wait()
            return carry

        lax.fori_loop(0, KWG, drain_idx, 0)

        def group(g, carry):
            gd = []
            for b in range(GB):
                j = GB * g + b
                gd.append(pltpu.async_copy(
                    table_hbm.at[idxv.at[j]], rbuf[b], semg[b]))
            cd = []
            for b in range(GB):
                j = GB * g + b
                gd[b].wait()
                cd.append(pltpu.async_copy(
                    rbuf[b], out_hbm.at[pl.ds((w + NW * j) * CHG, CHG)], semc[b]))
            for b in range(GB):
                cd[b].wait()
            return carry

        lax.fori_loop(0, KWG // GB, group, 0)

        @pl.when(w < TAILG)
        def _():
            c = KWG * NW + w
            pltpu.sync_copy(idx_hbm.at[c], idxv.at[KWG])
            pltpu.async_copy(table_hbm.at[idxv.at[KWG]], r0, sg0).wait()
            pltpu.sync_copy(r0, out_hbm.at[pl.ds(c * CHG, CHG)])

    return gk(table, idx2)


def _sc_scatter(vals, idx2):
    NRs = NR_NODE // NS
    zeros = jnp.zeros((NRs, D), jnp.float32)
    mesh = plsc.VectorSubcoreMesh(core_axis_name="c", subcore_axis_name="s")

    @functools.partial(
        pl.kernel,
        out_type=jax.ShapeDtypeStruct((NC, NR_NODE, D), jnp.float32),
        mesh=mesh,
        scratch_types=[
            pltpu.VMEM_SHARED((NR_NODE, D), jnp.float32),
            pltpu.VMEM((KWS + 1, CHS), jnp.int32),
            pltpu.VMEM((CHS, D), jnp.float32),
            pltpu.VMEM((CHS, D), jnp.float32),
            pltpu.VMEM((CHS, D), jnp.float32),
            pltpu.SemaphoreType.DMA,
            pltpu.SemaphoreType.DMA,
            pltpu.SemaphoreType.DMA,
            pltpu.SemaphoreType.DMA,
            pltpu.SemaphoreType.DMA,
            pltpu.SemaphoreType.DMA,
            pltpu.SemaphoreType.DMA,
        ],
    )
    def sk(vals_hbm, idx_hbm, zeros_hbm, out_hbm, acc, idxv, v0, v1, v2,
           semi, sl0, sl1, sl2, ss0, ss1, ss2):
        vbuf = (v0, v1, v2)
        seml = (sl0, sl1, sl2)
        sems = (ss0, ss1, ss2)
        c = lax.axis_index("c")
        s = lax.axis_index("s")
        w = s * NC + c
        pltpu.sync_copy(zeros_hbm, acc.at[pl.ds(s * NRs, NRs)])

        def fire_idx(j, carry):
            pltpu.async_copy(idx_hbm.at[w + NW * j], idxv.at[j], semi)
            return carry

        lax.fori_loop(0, KWS, fire_idx, 0)

        def drain_idx(j, carry):
            pltpu.make_async_copy(idx_hbm.at[0], idxv.at[0], semi).wait()
            return carry

        lax.fori_loop(0, KWS, drain_idx, 0)
        plsc.subcore_barrier()

        def group(g, carry):
            ld = []
            for b in range(GB):
                j = GB * g + b
                ld.append(pltpu.async_copy(
                    vals_hbm.at[pl.ds((w + NW * j) * CHS, CHS)], vbuf[b], seml[b]))
            sd = []
            for b in range(GB):
                j = GB * g + b
                ld[b].wait()
                sd.append(pltpu.async_copy(
                    vbuf[b], acc.at[idxv.at[j]], sems[b], add=True))
            for b in range(GB):
                sd[b].wait()
            return carry

        lax.fori_loop(0, KWS // GB, group, 0)

        @pl.when(w < TAILS)
        def _():
            ct = KWS * NW + w
            pltpu.sync_copy(idx_hbm.at[ct], idxv.at[KWS])
            pltpu.sync_copy(vals_hbm.at[pl.ds(ct * CHS, CHS)], v0)
            pltpu.sync_copy(v0, acc.at[idxv.at[KWS]], add=True)

        plsc.subcore_barrier()
        pltpu.sync_copy(acc.at[pl.ds(s * NRs, NRs)],
                        out_hbm.at[c, pl.ds(s * NRs, NRs)])

    return sk(vals, idx2, zeros)


def _sph(vec):
    r = jnp.linalg.norm(vec, axis=-1, keepdims=True)
    u = vec / (r + 1e-8)
    x, y, z = u[:, 0], u[:, 1], u[:, 2]
    s3 = 3.0 ** 0.5
    s5 = 5.0 ** 0.5
    s15 = 15.0 ** 0.5
    return jnp.stack([jnp.ones_like(x), s3 * x, s3 * y, s3 * z,
                      s15 * x * y, s15 * y * z, (s5 / 2.0) * (3.0 * z * z - 1.0),
                      s15 * x * z, (s15 / 2.0) * (x * x - y * y)], axis=-1)


def _rbf(r):
    centers = jnp.linspace(0.0, CUT, NB)
    width = CUT / NB
    return jnp.exp(-(((r[:, None] - centers[None, :]) / width) ** 2))


BN = 1000
NGP = 560


def _head_body(x_ref, wproj_ref, h1_ref, h2_ref, o_ref):
    x = x_ref[...]
    xp = jnp.dot(x, wproj_ref[...], preferred_element_type=jnp.float32)
    mu = xp.mean(axis=-1, keepdims=True)
    sd = jnp.sqrt(((xp - mu) ** 2).mean(axis=-1, keepdims=True) + 1e-5)
    xn = (xp - mu) / sd
    h = jnp.dot(xn, h1_ref[...], preferred_element_type=jnp.float32)
    h = h * jax.nn.sigmoid(h)
    o_ref[...] = jnp.dot(h, h2_ref[...], preferred_element_type=jnp.float32)


def _head(x, Wproj, head_w1, head_w2p):
    return pl.pallas_call(
        _head_body,
        grid=(N // BN,),
        in_specs=[
            pl.BlockSpec((BN, D), lambda i: (i, 0)),
            pl.BlockSpec((D, DF), lambda i: (0, 0)),
            pl.BlockSpec((DF, DF), lambda i: (0, 0)),
            pl.BlockSpec((DF, 128), lambda i: (0, 0)),
        ],
        out_specs=pl.BlockSpec((BN, 128), lambda i: (i, 0)),
        out_shape=jax.ShapeDtypeStruct((N, 128), jnp.float32),
    )(x, Wproj, head_w1, head_w2p)


def _pool_body(b_ref, x_ref, o_ref):
    i = pl.program_id(0)
    bb = b_ref[0, 0, :]
    oh = (lax.broadcasted_iota(jnp.int32, (NGP, BN), 0) == bb[None, :]
          ).astype(jnp.float32)
    acc = jnp.dot(oh, x_ref[...], preferred_element_type=jnp.float32)

    @pl.when(i == 0)
    def _():
        o_ref[...] = acc

    @pl.when(i > 0)
    def _():
        o_ref[...] += acc


def _pool(batch, node_out):
    b3 = batch.astype(jnp.int32).reshape(N // BN, 1, BN)
    return pl.pallas_call(
        _pool_body,
        grid=(N // BN,),
        in_specs=[
            pl.BlockSpec((1, 1, BN), lambda i: (i, 0, 0)),
            pl.BlockSpec((BN, D), lambda i: (i, 0)),
        ],
        out_specs=pl.BlockSpec((NGP, D), lambda i: (0, 0)),
        out_shape=jax.ShapeDtypeStruct((NGP, D), jnp.float32),
    )(b3, node_out)


def kernel(f_in, pos, batch, node_atom, edge_src, edge_dst, atom_table,
           deg_w1, deg_w2, deg_w3, Wq, Wk, Wv, Wo, We1, We2, Wsh, Wf1, Wf2,
           Wproj, head_w1, head_w2):
    src2 = edge_src.astype(jnp.int32).reshape(NCHG, CHG)
    dst2 = edge_dst.astype(jnp.int32).reshape(NCHG, CHG)
    dst2s = edge_dst.astype(jnp.int32).reshape(NCHS, CHS)

    pos_p = jnp.pad(pos, ((0, 0), (0, 125)))
    ps = _sc_gather(pos_p, src2)
    pd = _sc_gather(pos_p, dst2)
    edge_vec = (ps - pd)[:, :3]
    sh = _sph(edge_vec)
    r = jnp.linalg.norm(edge_vec, axis=-1)
    rbf = _rbf(r)

    x0 = (f_in[:, None] == jnp.arange(60)[None, :]).astype(jnp.float32) @ atom_table

    g = jax.nn.silu(rbf @ deg_w1)
    g = jax.nn.silu(g @ deg_w2)
    gate = g @ deg_w3
    x0s = _sc_gather(x0, src2)
    msg = x0s * gate
    p = _sc_scatter(msg, dst2s)
    deg_emb = (p[0] + p[1])[:N] / (AVG_DEG ** 0.5)
    x = x0 + deg_emb

    for l in range(L):
        ef = jax.nn.silu(rbf @ We1[l]) @ We2[l] + sh @ Wsh[l]
        xq = x @ Wq[l]
        qd = _sc_gather(xq, dst2)
        xs = _sc_gather(x, src2)
        src_f = xs + ef
        k = src_f @ Wk[l]
        v = src_f @ Wv[l]
        logits = (qd.reshape(E, H, DH) * k.reshape(E, H, DH)).sum(-1) / (DH ** 0.5)
        gm = logits.max(axis=0)
        a = jnp.exp(logits - gm[None, :])
        av = (a[:, :, None] * v.reshape(E, H, DH)).reshape(E, D)
        p = _sc_scatter(av, dst2s)
        pa = _sc_scatter(jnp.pad(a, ((0, 0), (0, D - H))), dst2s)
        S = (p[0] + p[1])[:N]
        den = (pa[0] + pa[1])[:N, :H].reshape(N, H, 1)
        agg = jnp.where(den > 0, S.reshape(N, H, DH) / jnp.where(den > 0, den, 1.0), 0.0)
        x = x + agg.reshape(N, D) @ Wo[l]
        x = x + jax.nn.silu(x @ Wf1[l]) @ Wf2[l]

    h2p = jnp.pad(head_w2, ((0, 0), (0, 127)))
    node_out = _head(x, Wproj, head_w1, h2p)

    out = _pool(batch, node_out)[:NG, :1] / (AVG_NODES ** 0.5)
    return out
